# Initial kernel scaffold; baseline (speedup 1.0000x reference)
import jax
import jax.numpy as jnp
from jax.experimental import pallas as pl


def kernel(x, W1, b1, g1, be1, m1, v1, Wg, bg, W2, b2, g2, be2, m2, v2):
    raise NotImplementedError("write your pallas kernel here")


# speedup vs baseline: 21.4376x; 21.4376x over previous
import functools

import jax
import jax.numpy as jnp
from jax import lax
from jax.experimental import pallas as pl
from jax.experimental.pallas import tpu as pltpu
from jax.experimental.pallas import tpu_sc as plsc

_B, _C, _H, _W = 4, 96, 56, 56
_N = _H * _W
_K = 9
_G = 4
_CG = (2 * _C) // _G
_R = 392
_NBLK = _N // _R
_TOT = _B * _N
_NW = 32
_NPW = _TOT // _NW
_CHUNK = 56
_NCH = _NPW // _CHUNK
_EPC = _CHUNK * _K
_ISUB = 4
_IPAD = _ISUB * 128
_CP = 128


def _k1_body(xt_ref, w1t_ref, b1_ref, m1_ref, iv1_ref, g1_ref, be1_ref,
             wa_ref, ba_ref, wd_ref, h_ref, a_ref, z_ref):
    x = xt_ref[0]
    h = jnp.dot(x, w1t_ref[...], preferred_element_type=jnp.float32)
    h = h + b1_ref[...]
    h = (h - m1_ref[...]) * iv1_ref[...] * g1_ref[...] + be1_ref[...]
    h_ref[0] = h
    a = jnp.dot(h, wa_ref[...], preferred_element_type=jnp.float32)
    a_ref[0] = jnp.maximum(a + ba_ref[...], 0.0)
    z = jnp.dot(h, wd_ref[...], preferred_element_type=jnp.float32)
    z_ref[0] = jnp.pad(z, ((0, 0), (0, _CP - _C)))


def _k1b_body(ht_ref, xnt_ref, sq_ref):
    hT = ht_ref[0]
    nrm = jnp.sqrt(jnp.sum(hT * hT, axis=0, keepdims=True))
    xnT = hT / jnp.maximum(nrm, 1e-12)
    xnt_ref[0] = xnT
    sq_ref[0] = jnp.sum(xnT * xnT, axis=0, keepdims=True)


def _k2_body(xn_ref, xnt_ref, sqr_ref, sqa_ref, nn_ref):
    b = pl.program_id(0)
    xr = xn_ref[0]
    xa = xnt_ref[0]
    sqr = sqr_ref[0]
    sqa = sqa_ref[0]
    dot = jnp.dot(xr, xa, preferred_element_type=jnp.float32)
    d = (sqr + sqa) - 2.0 * dot
    cols = lax.broadcasted_iota(jnp.int32, (_R, _N), 1)
    inf = jnp.float32(jnp.inf)
    picks = []
    for _ in range(_K):
        m = jnp.min(d, axis=1, keepdims=True)
        cand = jnp.where(d == m, cols, _N)
        sel = jnp.min(cand, axis=1, keepdims=True)
        picks.append(sel[:, 0])
        d = jnp.where(cols == sel, inf, d)
    nn_ref[0] = jnp.stack(picks, axis=1) + b * _N


def _k4_body(a_ref, bm_ref, xt_ref, w2a_ref, w2b_ref, b2_ref, y_ref):
    y = jnp.dot(a_ref[0], w2a_ref[...], preferred_element_type=jnp.float32)
    y = y + jnp.dot(bm_ref[0], w2b_ref[...], preferred_element_type=jnp.float32)
    y_ref[0] = y + b2_ref[...] + xt_ref[0]


@functools.cache
def _sc_kernel():
    mesh = plsc.VectorSubcoreMesh(core_axis_name="c", subcore_axis_name="s")

    @functools.partial(
        pl.kernel,
        mesh=mesh,
        out_type=jax.ShapeDtypeStruct((_TOT, _CP), jnp.float32),
        scratch_types=[
            pltpu.VMEM((_ISUB, 128), jnp.int32),
            pltpu.VMEM((_IPAD, _CP), jnp.float32),
            pltpu.VMEM((_CHUNK, _CP), jnp.float32),
            pltpu.VMEM((_CHUNK, _CP), jnp.float32),
            pltpu.VMEM((1, _CP), jnp.float32),
            pltpu.SemaphoreType.DMA,
        ],
    )
    def body(z_hbm, idx_hbm, bd_hbm, out_hbm,
             idx_v, rows_v, ctr_v, out_v, bd_v, sem):
        wid = lax.axis_index("s") * 2 + lax.axis_index("c")
        pltpu.sync_copy(bd_hbm, bd_v)

        def chunk(cix, carry):
            base = pl.multiple_of(wid * _NPW + cix * _CHUNK, _CHUNK)
            pltpu.sync_copy(idx_hbm.at[wid * _NCH + cix], idx_v)
            cps = [
                pltpu.async_copy(z_hbm.at[idx_v.at[u]],
                                 rows_v.at[pl.ds(u * 128, 128)], sem)
                for u in range(_ISUB)
            ]
            for cp in cps:
                cp.wait()
            pltpu.sync_copy(z_hbm.at[pl.ds(base, _CHUNK)], ctr_v)

            def node(n, c2):
                e = n * _K
                for j in range(_CP // 16):
                    sl = pl.ds(j * 16, 16)
                    m = rows_v[e, sl]
                    for k in range(1, _K):
                        m = jnp.maximum(m, rows_v[e + k, sl])
                    out_v[n, sl] = jnp.maximum(
                        m - ctr_v[n, sl] + bd_v[0, sl], 0.0)
                return c2

            lax.fori_loop(0, _CHUNK, node, 0)
            pltpu.sync_copy(out_v, out_hbm.at[pl.ds(base, _CHUNK)])
            return carry

        lax.fori_loop(0, _NCH, chunk, 0)

    return body


def _sc_gather_max(zf, idx3, bd):
    return _sc_kernel()(zf, idx3, bd)


def kernel(x, W1, b1, g1, be1, m1, v1, Wg, bg, W2, b2, g2, be2, m2, v2):
    f32 = jnp.float32
    w1t = W1.T
    iv1 = (1.0 / jnp.sqrt(v1 + 1e-5))[None, :]
    wgr = Wg.reshape(_G, _CG, _CG)
    z96 = jnp.zeros((_C, _C), f32)
    wa = z96.at[:_CG, :_CG].set(wgr[0].T).at[_CG:, _CG:].set(wgr[1].T)
    wd = z96.at[:_CG, :_CG].set(wgr[2].T).at[_CG:, _CG:].set(wgr[3].T)
    ba = bg[:_C][None, :]
    bd = bg[_C:][None, :]
    s2 = g2 / jnp.sqrt(v2 + 1e-5)
    w2f = W2 * s2[:, None]
    w2a = w2f[:, :_C].T
    w2b = w2f[:, _C:].T
    b2f = ((b2 - m2) * s2 + be2)[None, :]

    xt = jnp.transpose(x.reshape(_B, _C, _N), (0, 2, 1))

    row_spec = pl.BlockSpec((1, _R, _C), lambda b, i: (b, i, 0))
    wspec = pl.BlockSpec((_C, _C), lambda b, i: (0, 0))
    bspec = pl.BlockSpec((1, _C), lambda b, i: (0, 0))
    fshape = jax.ShapeDtypeStruct((_B, _N, _C), f32)

    zspec = pl.BlockSpec((1, _R, _CP), lambda b, i: (b, i, 0))
    zshape = jax.ShapeDtypeStruct((_B, _N, _CP), f32)
    h, a, z = pl.pallas_call(
        _k1_body,
        grid=(_B, _NBLK),
        in_specs=[row_spec, wspec] + [bspec] * 5 + [wspec, bspec, wspec],
        out_specs=[row_spec, row_spec, zspec],
        out_shape=[fshape, fshape, zshape],
    )(xt, w1t, b1[None], m1[None], iv1, g1[None], be1[None], wa, ba, wd)

    ht = jnp.transpose(h, (0, 2, 1))
    cspec = pl.BlockSpec((1, _C, _N), lambda b: (b, 0, 0))
    sspec = pl.BlockSpec((1, 1, _N), lambda b: (b, 0, 0))
    xnt, sq = pl.pallas_call(
        _k1b_body,
        grid=(_B,),
        in_specs=[cspec],
        out_specs=[cspec, sspec],
        out_shape=[jax.ShapeDtypeStruct((_B, _C, _N), f32),
                   jax.ShapeDtypeStruct((_B, 1, _N), f32)],
    )(ht)

    xn = jnp.transpose(xnt, (0, 2, 1))
    sqt = jnp.transpose(sq, (0, 2, 1))

    nn = pl.pallas_call(
        _k2_body,
        grid=(_B, _NBLK),
        in_specs=[row_spec,
                  pl.BlockSpec((1, _C, _N), lambda b, i: (b, 0, 0)),
                  pl.BlockSpec((1, _R, 1), lambda b, i: (b, i, 0)),
                  pl.BlockSpec((1, 1, _N), lambda b, i: (b, 0, 0))],
        out_specs=pl.BlockSpec((1, _R, _K), lambda b, i: (b, i, 0)),
        out_shape=jax.ShapeDtypeStruct((_B, _N, _K), jnp.int32),
    )(xn, xnt, sqt, sq)

    idx3 = jnp.pad(nn.reshape(_NW * _NCH, _EPC),
                   ((0, 0), (0, _IPAD - _EPC)))
    idx3 = idx3.reshape(_NW * _NCH, _ISUB, 128)
    bd_p = jnp.pad(bd, ((0, 0), (0, _CP - _C)))
    outb = _sc_gather_max(z.reshape(_TOT, _CP), idx3, bd_p)

    w2b_p = jnp.pad(w2b, ((0, _CP - _C), (0, 0)))
    yt = pl.pallas_call(
        _k4_body,
        grid=(_B, _NBLK),
        in_specs=[row_spec,
                  pl.BlockSpec((1, _R, _CP), lambda b, i: (b, i, 0)),
                  row_spec,
                  wspec,
                  pl.BlockSpec((_CP, _C), lambda b, i: (0, 0)),
                  bspec],
        out_specs=row_spec,
        out_shape=fshape,
    )(a, outb.reshape(_B, _N, _CP), xt, w2a, w2b_p, b2f)

    return jnp.transpose(yt, (0, 2, 1)).reshape(_B, _C, _H, _W)


# speedup vs baseline: 21.7133x; 1.0129x over previous
import functools

import jax
import jax.numpy as jnp
from jax import lax
from jax.experimental import pallas as pl
from jax.experimental.pallas import tpu as pltpu
from jax.experimental.pallas import tpu_sc as plsc

_B, _C, _H, _W = 4, 96, 56, 56
_N = _H * _W
_K = 9
_G = 4
_CG = (2 * _C) // _G
_R = 392
_NBLK = _N // _R
_TOT = _B * _N
_NW = 32
_NPW = _TOT // _NW
_CHUNK = 56
_NCH = _NPW // _CHUNK
_EPC = _CHUNK * _K
_ISUB = 4
_IPAD = _ISUB * 128
_CP = 128
_N2 = 3200
_NSUB = _N2 // 128


def _k1_body(xt_ref, w1t_ref, b1_ref, m1_ref, iv1_ref, g1_ref, be1_ref,
             wa_ref, ba_ref, wd_ref, h_ref, a_ref, z_ref):
    x = xt_ref[0]
    h = jnp.dot(x, w1t_ref[...], preferred_element_type=jnp.float32)
    h = h + b1_ref[...]
    h = (h - m1_ref[...]) * iv1_ref[...] * g1_ref[...] + be1_ref[...]
    h_ref[0] = h
    a = jnp.dot(h, wa_ref[...], preferred_element_type=jnp.float32)
    a_ref[0] = jnp.maximum(a + ba_ref[...], 0.0)
    z = jnp.dot(h, wd_ref[...], preferred_element_type=jnp.float32)
    z_ref[0] = jnp.pad(z, ((0, 0), (0, _CP - _C)))


def _k1b_body(ht_ref, xnt_ref, sq_ref):
    hT = ht_ref[0]
    nrm = jnp.sqrt(jnp.sum(hT * hT, axis=0, keepdims=True))
    xnT = hT / jnp.maximum(nrm, 1e-12)
    xnt_ref[0] = xnT
    sq_ref[0] = jnp.sum(xnT * xnT, axis=0, keepdims=True)


def _k2_body(xn_ref, xnt_ref, sqr_ref, sqa_ref, nn_ref):
    b = pl.program_id(0)
    xr = xn_ref[0]
    xa = xnt_ref[0]
    sqr = sqr_ref[0]
    sqa = sqa_ref[0]
    dot = jnp.dot(xr, xa, preferred_element_type=jnp.float32)
    d = (sqr + sqa) - 2.0 * dot
    blocks = [d[:, g * 128:(g + 1) * 128] for g in range(_NSUB)]
    lane = lax.broadcasted_iota(jnp.int32, (_R, 128), 1)
    inf = jnp.float32(jnp.inf)
    picks = []
    for r in range(_K):
        bm = blocks[0]
        for g in range(1, _NSUB):
            bm = jnp.minimum(bm, blocks[g])
        m = jnp.min(bm, axis=1, keepdims=True)
        ci = None
        for g in range(_NSUB):
            cand = jnp.where(blocks[g] == m, lane + g * 128, _N2)
            ci = cand if ci is None else jnp.minimum(ci, cand)
        sel = jnp.min(ci, axis=1, keepdims=True)
        picks.append(sel[:, 0])
        if r + 1 < _K:
            for g in range(_NSUB):
                blocks[g] = jnp.where(lane + g * 128 == sel, inf, blocks[g])
    nn_ref[0] = jnp.stack(picks, axis=1) + b * _N


def _k4_body(a_ref, bm_ref, xt_ref, w2a_ref, w2b_ref, b2_ref, y_ref):
    y = jnp.dot(a_ref[0], w2a_ref[...], preferred_element_type=jnp.float32)
    y = y + jnp.dot(bm_ref[0], w2b_ref[...], preferred_element_type=jnp.float32)
    y_ref[0] = y + b2_ref[...] + xt_ref[0]


@functools.cache
def _sc_kernel():
    mesh = plsc.VectorSubcoreMesh(core_axis_name="c", subcore_axis_name="s")

    @functools.partial(
        pl.kernel,
        mesh=mesh,
        out_type=jax.ShapeDtypeStruct((_TOT, _CP), jnp.float32),
        scratch_types=[
            pltpu.VMEM((_ISUB, 128), jnp.int32),
            pltpu.VMEM((_IPAD, _CP), jnp.float32),
            pltpu.VMEM((_CHUNK, _CP), jnp.float32),
            pltpu.VMEM((_CHUNK, _CP), jnp.float32),
            pltpu.VMEM((1, _CP), jnp.float32),
            pltpu.SemaphoreType.DMA,
        ],
    )
    def body(z_hbm, idx_hbm, bd_hbm, out_hbm,
             idx_v, rows_v, ctr_v, out_v, bd_v, sem):
        wid = lax.axis_index("s") * 2 + lax.axis_index("c")
        pltpu.sync_copy(bd_hbm, bd_v)

        def chunk(cix, carry):
            base = pl.multiple_of(wid * _NPW + cix * _CHUNK, _CHUNK)
            pltpu.sync_copy(idx_hbm.at[wid * _NCH + cix], idx_v)
            cps = [
                pltpu.async_copy(z_hbm.at[idx_v.at[u]],
                                 rows_v.at[pl.ds(u * 128, 128)], sem)
                for u in range(_ISUB)
            ]
            for cp in cps:
                cp.wait()
            pltpu.sync_copy(z_hbm.at[pl.ds(base, _CHUNK)], ctr_v)

            def node(n, c2):
                e = n * _K
                for j in range(_CP // 16):
                    sl = pl.ds(j * 16, 16)
                    m = rows_v[e, sl]
                    for k in range(1, _K):
                        m = jnp.maximum(m, rows_v[e + k, sl])
                    out_v[n, sl] = jnp.maximum(
                        m - ctr_v[n, sl] + bd_v[0, sl], 0.0)
                return c2

            lax.fori_loop(0, _CHUNK, node, 0)
            pltpu.sync_copy(out_v, out_hbm.at[pl.ds(base, _CHUNK)])
            return carry

        lax.fori_loop(0, _NCH, chunk, 0)

    return body


def _sc_gather_max(zf, idx3, bd):
    return _sc_kernel()(zf, idx3, bd)


def kernel(x, W1, b1, g1, be1, m1, v1, Wg, bg, W2, b2, g2, be2, m2, v2):
    f32 = jnp.float32
    w1t = W1.T
    iv1 = (1.0 / jnp.sqrt(v1 + 1e-5))[None, :]
    wgr = Wg.reshape(_G, _CG, _CG)
    z96 = jnp.zeros((_C, _C), f32)
    wa = z96.at[:_CG, :_CG].set(wgr[0].T).at[_CG:, _CG:].set(wgr[1].T)
    wd = z96.at[:_CG, :_CG].set(wgr[2].T).at[_CG:, _CG:].set(wgr[3].T)
    ba = bg[:_C][None, :]
    bd = bg[_C:][None, :]
    s2 = g2 / jnp.sqrt(v2 + 1e-5)
    w2f = W2 * s2[:, None]
    w2a = w2f[:, :_C].T
    w2b = w2f[:, _C:].T
    b2f = ((b2 - m2) * s2 + be2)[None, :]

    xt = jnp.transpose(x.reshape(_B, _C, _N), (0, 2, 1))

    row_spec = pl.BlockSpec((1, _R, _C), lambda b, i: (b, i, 0))
    wspec = pl.BlockSpec((_C, _C), lambda b, i: (0, 0))
    bspec = pl.BlockSpec((1, _C), lambda b, i: (0, 0))
    fshape = jax.ShapeDtypeStruct((_B, _N, _C), f32)

    zspec = pl.BlockSpec((1, _R, _CP), lambda b, i: (b, i, 0))
    zshape = jax.ShapeDtypeStruct((_B, _N, _CP), f32)
    h, a, z = pl.pallas_call(
        _k1_body,
        grid=(_B, _NBLK),
        in_specs=[row_spec, wspec] + [bspec] * 5 + [wspec, bspec, wspec],
        out_specs=[row_spec, row_spec, zspec],
        out_shape=[fshape, fshape, zshape],
    )(xt, w1t, b1[None], m1[None], iv1, g1[None], be1[None], wa, ba, wd)

    ht = jnp.transpose(h, (0, 2, 1))
    cspec = pl.BlockSpec((1, _C, _N), lambda b: (b, 0, 0))
    sspec = pl.BlockSpec((1, 1, _N), lambda b: (b, 0, 0))
    xnt, sq = pl.pallas_call(
        _k1b_body,
        grid=(_B,),
        in_specs=[cspec],
        out_specs=[cspec, sspec],
        out_shape=[jax.ShapeDtypeStruct((_B, _C, _N), f32),
                   jax.ShapeDtypeStruct((_B, 1, _N), f32)],
    )(ht)

    xn = jnp.transpose(xnt, (0, 2, 1))
    sqt = jnp.transpose(sq, (0, 2, 1))
    xnt_p = jnp.pad(xnt, ((0, 0), (0, 0), (0, _N2 - _N)))
    sq_p = jnp.pad(sq, ((0, 0), (0, 0), (0, _N2 - _N)),
                   constant_values=jnp.inf)

    nn = pl.pallas_call(
        _k2_body,
        grid=(_B, _NBLK),
        in_specs=[row_spec,
                  pl.BlockSpec((1, _C, _N2), lambda b, i: (b, 0, 0)),
                  pl.BlockSpec((1, _R, 1), lambda b, i: (b, i, 0)),
                  pl.BlockSpec((1, 1, _N2), lambda b, i: (b, 0, 0))],
        out_specs=pl.BlockSpec((1, _R, _K), lambda b, i: (b, i, 0)),
        out_shape=jax.ShapeDtypeStruct((_B, _N, _K), jnp.int32),
    )(xn, xnt_p, sqt, sq_p)

    idx3 = jnp.pad(nn.reshape(_NW * _NCH, _EPC),
                   ((0, 0), (0, _IPAD - _EPC)))
    idx3 = idx3.reshape(_NW * _NCH, _ISUB, 128)
    bd_p = jnp.pad(bd, ((0, 0), (0, _CP - _C)))
    outb = _sc_gather_max(z.reshape(_TOT, _CP), idx3, bd_p)

    w2b_p = jnp.pad(w2b, ((0, _CP - _C), (0, 0)))
    yt = pl.pallas_call(
        _k4_body,
        grid=(_B, _NBLK),
        in_specs=[row_spec,
                  pl.BlockSpec((1, _R, _CP), lambda b, i: (b, i, 0)),
                  row_spec,
                  wspec,
                  pl.BlockSpec((_CP, _C), lambda b, i: (0, 0)),
                  bspec],
        out_specs=row_spec,
        out_shape=fshape,
    )(a, outb.reshape(_B, _N, _CP), xt, w2a, w2b_p, b2f)

    return jnp.transpose(yt, (0, 2, 1)).reshape(_B, _C, _H, _W)


# speedup vs baseline: 21.7205x; 1.0003x over previous
import functools

import jax
import jax.numpy as jnp
from jax import lax
from jax.experimental import pallas as pl
from jax.experimental.pallas import tpu as pltpu
from jax.experimental.pallas import tpu_sc as plsc

_B, _C, _H, _W = 4, 96, 56, 56
_N = _H * _W
_K = 9
_G = 4
_CG = (2 * _C) // _G
_R = 392
_NBLK = _N // _R
_TOT = _B * _N
_NW = 32
_NPW = _TOT // _NW
_CHUNK = 56
_NCH = _NPW // _CHUNK
_EPC = _CHUNK * _K
_ISUB = 4
_IPAD = _ISUB * 128
_CP = 128
_N2 = 3200
_NSUB = _N2 // 128


def _k1_body(xt_ref, w1t_ref, b1_ref, m1_ref, iv1_ref, g1_ref, be1_ref,
             wa_ref, ba_ref, wd_ref, h_ref, a_ref, z_ref):
    x = xt_ref[0]
    h = jnp.dot(x, w1t_ref[...], preferred_element_type=jnp.float32)
    h = h + b1_ref[...]
    h = (h - m1_ref[...]) * iv1_ref[...] * g1_ref[...] + be1_ref[...]
    h_ref[0] = h
    a = jnp.dot(h, wa_ref[...], preferred_element_type=jnp.float32)
    a_ref[0] = jnp.maximum(a + ba_ref[...], 0.0)
    z = jnp.dot(h, wd_ref[...], preferred_element_type=jnp.float32)
    z_ref[0] = jnp.pad(z, ((0, 0), (0, _CP - _C)))


def _k1b_body(ht_ref, xnt_ref, sq_ref):
    hT = ht_ref[0]
    nrm = jnp.sqrt(jnp.sum(hT * hT, axis=0, keepdims=True))
    xnT = hT / jnp.maximum(nrm, 1e-12)
    xnt_ref[0] = xnT
    sq_ref[0] = jnp.sum(xnT * xnT, axis=0, keepdims=True)


def _k2_body(xn_ref, xnt_ref, sqr_ref, sqa_ref, nn_ref):
    b = pl.program_id(0)
    xr = xn_ref[0]
    xa = xnt_ref[0]
    sqr = sqr_ref[0]
    sqa = sqa_ref[0]
    dot = jnp.dot(xr, xa, preferred_element_type=jnp.float32)
    d = (sqr + sqa) - 2.0 * dot
    blocks = [d[:, g * 128:(g + 1) * 128] for g in range(_NSUB)]
    lane = lax.broadcasted_iota(jnp.int32, (_R, 128), 1)
    inf = jnp.float32(jnp.inf)
    picks = []
    for r in range(_K):
        bm = blocks[0]
        for g in range(1, _NSUB):
            bm = jnp.minimum(bm, blocks[g])
        m = jnp.min(bm, axis=1, keepdims=True)
        ci = None
        for g in range(_NSUB):
            cand = jnp.where(blocks[g] == m, lane + g * 128, _N2)
            ci = cand if ci is None else jnp.minimum(ci, cand)
        sel = jnp.min(ci, axis=1, keepdims=True)
        picks.append(sel[:, 0])
        if r + 1 < _K:
            for g in range(_NSUB):
                blocks[g] = jnp.where(lane + g * 128 == sel, inf, blocks[g])
    nn_ref[0] = jnp.stack(picks, axis=1) + b * _N


def _k4_body(a_ref, bm_ref, xt_ref, w2a_ref, w2b_ref, b2_ref, y_ref):
    y = jnp.dot(a_ref[0], w2a_ref[...], preferred_element_type=jnp.float32)
    y = y + jnp.dot(bm_ref[0], w2b_ref[...], preferred_element_type=jnp.float32)
    y_ref[0] = y + b2_ref[...] + xt_ref[0]


@functools.cache
def _sc_kernel():
    mesh = plsc.VectorSubcoreMesh(core_axis_name="c", subcore_axis_name="s")

    @functools.partial(
        pl.kernel,
        mesh=mesh,
        out_type=jax.ShapeDtypeStruct((_TOT, _CP), jnp.float32),
        scratch_types=[
            pltpu.VMEM((_ISUB, 128), jnp.int32),
            pltpu.VMEM((_IPAD, _CP), jnp.float32),
            pltpu.VMEM((_CHUNK, _CP), jnp.float32),
            pltpu.VMEM((_CHUNK, _CP), jnp.float32),
            pltpu.VMEM((1, _CP), jnp.float32),
            pltpu.SemaphoreType.DMA,
        ],
    )
    def body(z_hbm, idx_hbm, bd_hbm, out_hbm,
             idx_v, rows_v, ctr_v, out_v, bd_v, sem):
        wid = lax.axis_index("s") * 2 + lax.axis_index("c")
        pltpu.sync_copy(bd_hbm, bd_v)

        def chunk(cix, carry):
            base = pl.multiple_of(wid * _NPW + cix * _CHUNK, _CHUNK)
            pltpu.sync_copy(idx_hbm.at[wid * _NCH + cix], idx_v)
            cps = [
                pltpu.async_copy(z_hbm.at[idx_v.at[u]],
                                 rows_v.at[pl.ds(u * 128, 128)], sem)
                for u in range(_ISUB)
            ]
            for cp in cps:
                cp.wait()
            pltpu.sync_copy(z_hbm.at[pl.ds(base, _CHUNK)], ctr_v)

            def node(n, c2):
                e = n * _K
                for j in range(_CP // 16):
                    sl = pl.ds(j * 16, 16)
                    m = rows_v[e, sl]
                    for k in range(1, _K):
                        m = jnp.maximum(m, rows_v[e + k, sl])
                    out_v[n, sl] = jnp.maximum(
                        m - ctr_v[n, sl] + bd_v[0, sl], 0.0)
                return c2

            lax.fori_loop(0, _CHUNK, node, 0)
            pltpu.sync_copy(out_v, out_hbm.at[pl.ds(base, _CHUNK)])
            return carry

        lax.fori_loop(0, _NCH, chunk, 0)

    return body


def _sc_gather_max(zf, idx3, bd):
    return _sc_kernel()(zf, idx3, bd)


def kernel(x, W1, b1, g1, be1, m1, v1, Wg, bg, W2, b2, g2, be2, m2, v2):
    f32 = jnp.float32
    w1t = W1.T
    iv1 = (1.0 / jnp.sqrt(v1 + 1e-5))[None, :]
    wgr = Wg.reshape(_G, _CG, _CG)
    z96 = jnp.zeros((_C, _C), f32)
    wa = z96.at[:_CG, :_CG].set(wgr[0].T).at[_CG:, _CG:].set(wgr[1].T)
    wd = z96.at[:_CG, :_CG].set(wgr[2].T).at[_CG:, _CG:].set(wgr[3].T)
    ba = bg[:_C][None, :]
    bd = bg[_C:][None, :]
    s2 = g2 / jnp.sqrt(v2 + 1e-5)
    w2f = W2 * s2[:, None]
    w2a = w2f[:, :_C].T
    w2b = w2f[:, _C:].T
    b2f = ((b2 - m2) * s2 + be2)[None, :]

    xt = jnp.transpose(x.reshape(_B, _C, _N), (0, 2, 1))

    row_spec = pl.BlockSpec((1, _R, _C), lambda b, i: (b, i, 0))
    wspec = pl.BlockSpec((_C, _C), lambda b, i: (0, 0))
    bspec = pl.BlockSpec((1, _C), lambda b, i: (0, 0))
    fshape = jax.ShapeDtypeStruct((_B, _N, _C), f32)

    zspec = pl.BlockSpec((1, _R, _CP), lambda b, i: (b, i, 0))
    zshape = jax.ShapeDtypeStruct((_B, _N, _CP), f32)
    h, a, z = pl.pallas_call(
        _k1_body,
        grid=(_B, _NBLK),
        in_specs=[row_spec, wspec] + [bspec] * 5 + [wspec, bspec, wspec],
        out_specs=[row_spec, row_spec, zspec],
        out_shape=[fshape, fshape, zshape],
    )(xt, w1t, b1[None], m1[None], iv1, g1[None], be1[None], wa, ba, wd)

    ht = jnp.transpose(h, (0, 2, 1))
    cspec = pl.BlockSpec((1, _C, _N), lambda b: (b, 0, 0))
    sspec = pl.BlockSpec((1, 1, _N), lambda b: (b, 0, 0))
    xnt, sq = pl.pallas_call(
        _k1b_body,
        grid=(_B,),
        in_specs=[cspec],
        out_specs=[cspec, sspec],
        out_shape=[jax.ShapeDtypeStruct((_B, _C, _N), f32),
                   jax.ShapeDtypeStruct((_B, 1, _N), f32)],
    )(ht)

    xn = jnp.transpose(xnt, (0, 2, 1))
    sqt = jnp.transpose(sq, (0, 2, 1))
    xnt_p = jnp.pad(xnt, ((0, 0), (0, 0), (0, _N2 - _N)))
    sq_p = jnp.pad(sq, ((0, 0), (0, 0), (0, _N2 - _N)),
                   constant_values=jnp.inf)

    nn = pl.pallas_call(
        _k2_body,
        grid=(_B, _NBLK),
        in_specs=[row_spec,
                  pl.BlockSpec((1, _C, _N2), lambda b, i: (b, 0, 0)),
                  pl.BlockSpec((1, _R, 1), lambda b, i: (b, i, 0)),
                  pl.BlockSpec((1, 1, _N2), lambda b, i: (b, 0, 0))],
        out_specs=pl.BlockSpec((1, _R, _K), lambda b, i: (b, i, 0)),
        out_shape=jax.ShapeDtypeStruct((_B, _N, _K), jnp.int32),
        compiler_params=pltpu.CompilerParams(
            dimension_semantics=("parallel", "parallel")),
    )(xn, xnt_p, sqt, sq_p)

    idx3 = jnp.pad(nn.reshape(_NW * _NCH, _EPC),
                   ((0, 0), (0, _IPAD - _EPC)))
    idx3 = idx3.reshape(_NW * _NCH, _ISUB, 128)
    bd_p = jnp.pad(bd, ((0, 0), (0, _CP - _C)))
    outb = _sc_gather_max(z.reshape(_TOT, _CP), idx3, bd_p)

    w2b_p = jnp.pad(w2b, ((0, _CP - _C), (0, 0)))
    yt = pl.pallas_call(
        _k4_body,
        grid=(_B, _NBLK),
        in_specs=[row_spec,
                  pl.BlockSpec((1, _R, _CP), lambda b, i: (b, i, 0)),
                  row_spec,
                  wspec,
                  pl.BlockSpec((_CP, _C), lambda b, i: (0, 0)),
                  bspec],
        out_specs=row_spec,
        out_shape=fshape,
    )(a, outb.reshape(_B, _N, _CP), xt, w2a, w2b_p, b2f)

    return jnp.transpose(yt, (0, 2, 1)).reshape(_B, _C, _H, _W)
